# baseline (speedup 1.0000x reference)
import functools
import math

import jax
import jax.numpy as jnp
from jax import lax
from jax.experimental import pallas as pl
from jax.experimental.pallas import tpu as pltpu
from jax.experimental.pallas import tpu_sc as plsc

N = 50000
F = 64
A = 8
H = 100
W = 100
NC = 2
NS = 16
NW = NC * NS
L = 16
GROUPS = 104
PTS_PER_W = GROUPS * L
NP = NW * PTS_PER_W
RING = 4
RW = 2 * F
OMEGA = 2.0 * math.pi / A
SIN_OMEGA = math.sin(OMEGA)


def _sinpoly(t):
    t2 = t * t
    return t * (1.0 + t2 * (-1.0 / 6.0 + t2 * (1.0 / 120.0 - t2 * (1.0 / 5040.0))))


@functools.partial(
    pl.kernel,
    out_type=jax.ShapeDtypeStruct((NP * F,), jnp.float32),
    mesh=plsc.VectorSubcoreMesh(
        core_axis_name="c", subcore_axis_name="s", num_cores=NC, num_subcores=NS
    ),
    scratch_types=[
        pltpu.VMEM((PTS_PER_W,), jnp.float32),
        pltpu.VMEM((PTS_PER_W,), jnp.float32),
        pltpu.VMEM((PTS_PER_W,), jnp.float32),
        pltpu.VMEM((16,), jnp.float32),
        pltpu.VMEM((GROUPS * 2 * L,), jnp.int32),
        pltpu.VMEM((GROUPS * 8 * L,), jnp.float32),
        [pltpu.VMEM((2 * L, RW), jnp.int32) for _ in range(RING)],
        [pltpu.VMEM((L * F,), jnp.float32) for _ in range(RING)],
        [pltpu.SemaphoreType.DMA for _ in range(RING)],
        [pltpu.SemaphoreType.DMA for _ in range(RING)],
    ],
    compiler_params=pltpu.CompilerParams(needs_layout_passes=False),
)
def _interp_sc(table, xs, ys, angs, az, out_hbm,
               x_v, y_v, a_v, az_v, idx_v, w_v, rows_s, out_s, gsem, osem):
    wid = lax.axis_index("s") * NC + lax.axis_index("c")
    base = pl.multiple_of(wid * PTS_PER_W, 8)
    pltpu.sync_copy(xs.at[pl.ds(base, PTS_PER_W)], x_v)
    pltpu.sync_copy(ys.at[pl.ds(base, PTS_PER_W)], y_v)
    pltpu.sync_copy(angs.at[pl.ds(base, PTS_PER_W)], a_v)
    pltpu.sync_copy(az, az_v)

    def dim_interp(v, n):
        cv = v.astype(jnp.int32)
        ceil = jnp.where(v > cv.astype(jnp.float32), cv + 1, cv)
        r = jnp.minimum(ceil, n - 1)
        lft = jnp.maximum(r - 1, 0)
        dl = jnp.maximum(v - lft.astype(jnp.float32), 0.0)
        dr = jnp.maximum(r.astype(jnp.float32) - v, 0.0)
        b0 = (dl == 0.0) & (dr == 0.0)
        dl = jnp.where(b0, 1.0, dl)
        dr = jnp.where(b0, 1.0, dr)
        return lft, r, dl, dr, dl + dr

    def phase_a(g, carry):
        off = g * L
        vx = x_v[pl.ds(off, L)]
        vy = y_v[pl.ds(off, L)]
        va = a_v[pl.ds(off, L)]

        l0, r0, dl0, dr0, den0 = dim_interp(vx, H)
        l1, r1, dl1, dr1, den1 = dim_interp(vy, W)

        t = (va + math.pi) * (1.0 / OMEGA)
        it = jnp.clip(t.astype(jnp.int32), 0, A - 1)
        tick = plsc.load_gather(az_v, [it])
        theta = va - tick
        inv = 1.0 / (den0 * den1 * SIN_OMEGA)
        s1 = _sinpoly(OMEGA - theta) * inv
        s2 = _sinpoly(theta) * inv

        wxl = jnp.where(r1 == 0, dr1 + dl1, dr1)
        wxr = jnp.where(r1 == 0, 0.0, dl1)

        ibase = g * (2 * L)
        wbase = g * (8 * L)
        for k, (yi, wy) in enumerate(((l0, dr0), (r0, dl0))):
            idx_v[pl.ds(ibase + k * L, L)] = (yi * W + l1) * A + it
            w_v[pl.ds(wbase + (4 * k + 0) * L, L)] = wy * wxl * s1
            w_v[pl.ds(wbase + (4 * k + 1) * L, L)] = wy * wxl * s2
            w_v[pl.ds(wbase + (4 * k + 2) * L, L)] = wy * wxr * s1
            w_v[pl.ds(wbase + (4 * k + 3) * L, L)] = wy * wxr * s2
        return carry

    lax.fori_loop(0, GROUPS, phase_a, 0)

    splat_ids = [jnp.full((L, 1), p, jnp.int32) for p in range(L)]
    _splat_dnums = lax.GatherDimensionNumbers(
        offset_dims=(), collapsed_slice_dims=(0,), start_index_map=(0,))

    def _splat(vec, pid):
        return lax.gather(vec, pid, _splat_dnums, (1,),
                          mode=lax.GatherScatterMode.PROMISE_IN_BOUNDS)

    def fire(g, s):
        pltpu.async_copy(
            table.at[idx_v.at[pl.ds(g * (2 * L), 2 * L)]],
            rows_s[s], gsem[s])

    def wait_rows(s):
        pltpu.make_async_copy(table.at[idx_v.at[pl.ds(0, 2 * L)]],
                              rows_s[s], gsem[s]).wait()

    def wait_out(s):
        pltpu.make_async_copy(out_s[s],
                              out_hbm.at[pl.ds(0, L * F)], osem[s]).wait()

    def accum(g, s):
        rows = rows_s[s]
        out = out_s[s]
        wb = g * (8 * L)
        w8 = [w_v[pl.ds(wb + j * L, L)] for j in range(8)]
        for p in range(L):
            ws = [_splat(w, splat_ids[p]) for w in w8]
            rid = [k * L + p for k in range(2)]
            ob = p * F
            acc = [None] * 4
            for k in range(2):
                for sl in range(4):
                    wv = ws[4 * k + sl]
                    for blk in range(2):
                        v = rows[rid[k], pl.ds(sl * 2 * L + blk * L, L)]
                        vb = plsc.bitcast(v, jnp.bfloat16)
                        a, b = plsc.unpack(vb, format=plsc.PackFormat.INTERLEAVED)
                        c0 = blk * 2
                        pa = a * wv
                        pb = b * wv
                        acc[c0] = pa if acc[c0] is None else acc[c0] + pa
                        acc[c0 + 1] = pb if acc[c0 + 1] is None else acc[c0 + 1] + pb
            for c in range(4):
                out[pl.ds(ob + c * L, L)] = acc[c]
        pltpu.async_copy(out, out_hbm.at[pl.ds(base * F + g * (L * F), L * F)],
                         osem[s])

    for s in range(RING):
        fire(s, s)

    def phase_b(j, carry):
        g0 = j * RING
        for s in range(RING):
            g = g0 + s
            wait_rows(s)

            @pl.when(j > 0)
            def _():
                wait_out(s)

            accum(g, s)
            gn = jnp.minimum(g + RING, GROUPS - 1)
            fire(gn, s)
        return carry

    lax.fori_loop(0, GROUPS // RING, phase_b, 0)
    for s in range(RING):
        wait_rows(s)
        wait_out(s)


def kernel(positions, angles, grid_values):
    x = positions[:, 0]
    y = positions[:, 1]
    pad = NP - N
    xs = jnp.pad(x, (0, pad))
    ys = jnp.pad(y, (0, pad))
    angs = jnp.pad(angles, (0, pad))
    t = jnp.transpose(grid_values, (2, 3, 1, 0))
    t = jnp.concatenate([t, jnp.roll(t, -1, axis=2)], -1)
    tx = jnp.concatenate([t[:, 1:], t[:, -1:]], axis=1)
    t = jnp.concatenate([t, tx], -1)
    t = t.reshape(H * W, A, 8, 2, L)
    t = jnp.swapaxes(t, -1, -2).reshape(H * W, A, 4 * F, 1)
    t = t.astype(jnp.bfloat16)
    table = lax.bitcast_convert_type(
        t.reshape(H * W * A, RW, 2), jnp.int32)
    az = jnp.linspace(-math.pi, math.pi, A + 1)[:-1].astype(jnp.float32)
    az16 = jnp.pad(az, (0, 16 - A))
    out = _interp_sc(table, xs, ys, angs, az16)
    return out.reshape(NP, F)[:N]
